# baseline (speedup 1.0000x reference)
import functools

import jax
import jax.numpy as jnp
from jax import lax
from jax.experimental import pallas as pl
from jax.experimental.pallas import tpu as pltpu
from jax.experimental.pallas import tpu_sc as plsc

N = 10000
E = 320000
D = 128
NUM_LAYERS = 4

_NC = 2
_NS = 16
_NW = _NC * _NS
_EPW = E // _NW
_CH = 80
_NCH = _EPW // _CH
_NP = 10240
_NPS = _NP // _NS

_sc_mesh = plsc.VectorSubcoreMesh(core_axis_name="c", subcore_axis_name="s")


def _deg_body(dst_hbm, ones_hbm, zeros_hbm, out_hbm, idx_v, ones_v, sem, acc_sh):
    c = lax.axis_index("c")
    s = lax.axis_index("s")
    wid = c * _NS + s
    rows = pl.ds(s * _NPS, _NPS)
    pltpu.sync_copy(zeros_hbm.at[rows], acc_sh.at[rows])
    pltpu.sync_copy(ones_hbm, ones_v)
    plsc.subcore_barrier()
    base = wid * _EPW

    def step(j, carry):
        pltpu.sync_copy(dst_hbm.at[pl.ds(base + j * _CH, _CH)], idx_v)
        pltpu.sync_copy(ones_v, acc_sh.at[idx_v], add=True)
        return carry

    lax.fori_loop(0, _NCH, step, 0)
    plsc.subcore_barrier()
    pltpu.sync_copy(acc_sh.at[rows], out_hbm.at[c, rows])


def _sc_degree(dst0):
    ones = jnp.ones((_CH, D), jnp.float32)
    zeros = jnp.zeros((_NP, D), jnp.float32)
    return pl.kernel(
        _deg_body,
        out_type=jax.ShapeDtypeStruct((_NC, _NP, D), jnp.float32),
        mesh=_sc_mesh,
        scratch_types=[
            pltpu.VMEM((_CH,), jnp.int32),
            pltpu.VMEM((_CH, D), jnp.float32),
            pltpu.SemaphoreType.DMA,
            pltpu.VMEM_SHARED((_NP, D), jnp.float32),
        ],
    )(dst0, ones, zeros)


def _scat_body(y_hbm, src_hbm, dst3_hbm, zeros_hbm, out_hbm,
               sidx_v, didx_v, rows0, rows1, sem0, sem1, acc_sh):
    c = lax.axis_index("c")
    s = lax.axis_index("s")
    wid = c * _NS + s
    rows = pl.ds(s * _NPS, _NPS)
    pltpu.sync_copy(zeros_hbm.at[rows], acc_sh.at[rows])
    pltpu.sync_copy(src_hbm.at[pl.ds(wid * _EPW, _EPW)], sidx_v)
    pltpu.sync_copy(dst3_hbm.at[wid], didx_v)
    plsc.subcore_barrier()

    def gstart(j, buf, sem):
        pltpu.async_copy(y_hbm.at[sidx_v.at[pl.ds(j * _CH, _CH)]], buf, sem)

    def gwait(j, buf, sem):
        pltpu.make_async_copy(
            y_hbm.at[sidx_v.at[pl.ds(j * _CH, _CH)]], buf, sem).wait()

    def scat(j, buf):
        pltpu.sync_copy(buf, acc_sh.at[didx_v.at[j]], add=True)

    gstart(0, rows0, sem0)

    def step(i, carry):
        j0 = 2 * i
        gstart(j0 + 1, rows1, sem1)
        gwait(j0, rows0, sem0)
        scat(j0, rows0)
        gstart(j0 + 2, rows0, sem0)
        gwait(j0 + 1, rows1, sem1)
        scat(j0 + 1, rows1)
        return carry

    lax.fori_loop(0, (_NCH - 1) // 2, step, 0)
    gwait(_NCH - 1, rows0, sem0)
    scat(_NCH - 1, rows0)
    plsc.subcore_barrier()
    pltpu.sync_copy(acc_sh.at[rows], out_hbm.at[c, rows])


def _sc_scatter(y, src0, dst3, zeros_nd):
    return pl.kernel(
        _scat_body,
        out_type=jax.ShapeDtypeStruct((_NC, _NP, D), jnp.float32),
        mesh=_sc_mesh,
        scratch_types=[
            pltpu.VMEM((_EPW,), jnp.int32),
            pltpu.VMEM((_NCH, _CH), jnp.int32),
            pltpu.VMEM((_CH, D), jnp.float32),
            pltpu.VMEM((_CH, D), jnp.float32),
            pltpu.SemaphoreType.DMA,
            pltpu.SemaphoreType.DMA,
            pltpu.VMEM_SHARED((_NP, D), jnp.float32),
        ],
    )(y, src0, dst3, zeros_nd)


def _gath_body(h_hbm, src_hbm, dst_hbm, hr_hbm, hc_hbm,
               sidx_v, didx_v, bufa0, bufa1, bufb0, bufb1,
               sema0, sema1, semb0, semb1):
    c = lax.axis_index("c")
    s = lax.axis_index("s")
    wid = c * _NS + s
    base = wid * _EPW
    pltpu.sync_copy(src_hbm.at[pl.ds(base, _EPW)], sidx_v)
    pltpu.sync_copy(dst_hbm.at[pl.ds(base, _EPW)], didx_v)

    def ga(j, buf, sem, idx_v):
        pltpu.async_copy(h_hbm.at[idx_v.at[pl.ds(j * _CH, _CH)]], buf, sem)

    def gw(j, buf, sem, idx_v):
        pltpu.make_async_copy(
            h_hbm.at[idx_v.at[pl.ds(j * _CH, _CH)]], buf, sem).wait()

    def wr(j, buf, out):
        pltpu.sync_copy(buf, out.at[pl.ds(base + j * _CH, _CH)])

    ga(0, bufa0, sema0, sidx_v)
    ga(0, bufb0, semb0, didx_v)

    def step(i, carry):
        j0 = 2 * i
        ga(j0 + 1, bufa1, sema1, sidx_v)
        ga(j0 + 1, bufb1, semb1, didx_v)
        gw(j0, bufa0, sema0, sidx_v)
        wr(j0, bufa0, hr_hbm)
        gw(j0, bufb0, semb0, didx_v)
        wr(j0, bufb0, hc_hbm)
        ga(j0 + 2, bufa0, sema0, sidx_v)
        ga(j0 + 2, bufb0, semb0, didx_v)
        gw(j0 + 1, bufa1, sema1, sidx_v)
        wr(j0 + 1, bufa1, hr_hbm)
        gw(j0 + 1, bufb1, semb1, didx_v)
        wr(j0 + 1, bufb1, hc_hbm)
        return carry

    lax.fori_loop(0, (_NCH - 1) // 2, step, 0)
    gw(_NCH - 1, bufa0, sema0, sidx_v)
    wr(_NCH - 1, bufa0, hr_hbm)
    gw(_NCH - 1, bufb0, semb0, didx_v)
    wr(_NCH - 1, bufb0, hc_hbm)


def _sc_edge_gather(h, src0, dst0):
    return pl.kernel(
        _gath_body,
        out_type=(
            jax.ShapeDtypeStruct((E, D), jnp.float32),
            jax.ShapeDtypeStruct((E, D), jnp.float32),
        ),
        mesh=_sc_mesh,
        scratch_types=[
            pltpu.VMEM((_EPW,), jnp.int32),
            pltpu.VMEM((_EPW,), jnp.int32),
            pltpu.VMEM((_CH, D), jnp.float32),
            pltpu.VMEM((_CH, D), jnp.float32),
            pltpu.VMEM((_CH, D), jnp.float32),
            pltpu.VMEM((_CH, D), jnp.float32),
            pltpu.SemaphoreType.DMA,
            pltpu.SemaphoreType.DMA,
            pltpu.SemaphoreType.DMA,
            pltpu.SemaphoreType.DMA,
        ],
    )(h, src0, dst0)


def _enc_body(cnt_ref, x_ref, w1_ref, b1_ref, w2_ref, b2_ref, w0_ref,
              h_ref, y_ref, dinv_ref):
    cnt = cnt_ref[...]
    deg = cnt[0, :N, 0:1] + cnt[1, :N, 0:1] + 1.0
    dinv = jax.lax.rsqrt(deg)
    dinv_ref[...] = dinv
    h = jnp.dot(x_ref[...], w1_ref[...], preferred_element_type=jnp.float32)
    h = jax.nn.relu(h + b1_ref[...])
    h = jnp.dot(h, w2_ref[...], preferred_element_type=jnp.float32) + b2_ref[...]
    h_ref[...] = h
    xw = jnp.dot(h, w0_ref[...], preferred_element_type=jnp.float32)
    y_ref[...] = xw * dinv


def _encoder(cnt, x, w1t, b1, w2t, b2, w0t):
    return pl.pallas_call(
        _enc_body,
        out_shape=(
            jax.ShapeDtypeStruct((N, D), jnp.float32),
            jax.ShapeDtypeStruct((N, D), jnp.float32),
            jax.ShapeDtypeStruct((N, 1), jnp.float32),
        ),
    )(cnt, x, w1t, b1, w2t, b2, w0t)


def _layer_body(has_res, has_next, *refs):
    it = iter(refs)
    parts = next(it)
    y = next(it); dinv = next(it)
    conv_b = next(it); bn_g = next(it); bn_b = next(it)
    if has_res:
        h_res = next(it); res_wt = next(it); res_b = next(it)
    if has_next:
        w_next = next(it)
    h_out = next(it)
    if has_next:
        y_next = next(it)

    p = parts[...]
    agg = (p[0, :N] + p[1, :N] + y[...]) * dinv[...] + conv_b[...]
    mu = jnp.mean(agg, axis=0, keepdims=True)
    var = jnp.mean((agg - mu) ** 2, axis=0, keepdims=True)
    hbn = (agg - mu) * jax.lax.rsqrt(var + 1e-5) * bn_g[...] + bn_b[...]
    h = jax.nn.relu(hbn)
    if has_res:
        h = h + jnp.dot(h_res[...], res_wt[...],
                        preferred_element_type=jnp.float32) + res_b[...]
    h_out[...] = h
    if has_next:
        y_next[...] = jnp.dot(h, w_next[...],
                              preferred_element_type=jnp.float32) * dinv[...]


def _layer(parts, y, dinv, conv_b, bn_g, bn_b, res=None, w_next=None):
    has_res = res is not None
    has_next = w_next is not None
    outs = [jax.ShapeDtypeStruct((N, D), jnp.float32)]
    if has_next:
        outs.append(jax.ShapeDtypeStruct((N, D), jnp.float32))
    args = [parts, y, dinv, conv_b, bn_g, bn_b]
    if has_res:
        args += list(res)
    if has_next:
        args.append(w_next)
    return pl.pallas_call(
        functools.partial(_layer_body, has_res, has_next),
        out_shape=tuple(outs),
    )(*args)


_EB = 2000


def _edge_body(hr_ref, hc_ref, ea_ref,
               ee_w1, ee_b1, ee_w2, ee_b2,
               ea_w1a, ea_w1b, ea_w1c, ea_b1, ea_w2, ea_b2,
               cl_w1a, cl_w1b, cl_b1, cl_w2, cl_b2, cl_w3, cl_b3,
               out_ref):
    bf = jnp.bfloat16
    f32 = jnp.float32
    hr = hr_ref[...]
    hc = hc_ref[...]
    ea = ea_ref[...]
    hrb = hr.astype(bf)
    hcb = hc.astype(bf)
    eab = ea.astype(bf)
    e = jax.nn.relu(jnp.dot(eab, ee_w1[...], preferred_element_type=f32)
                    + ee_b1[...])
    e = jnp.dot(e.astype(bf), ee_w2[...], preferred_element_type=f32) + ee_b2[...]
    a = (jnp.dot(hrb, ea_w1a[...], preferred_element_type=f32)
         + jnp.dot(hcb, ea_w1b[...], preferred_element_type=f32)
         + jnp.dot(eab, ea_w1c[...], preferred_element_type=f32) + ea_b1[...])
    a = jax.nn.relu(a)
    w = jax.nn.sigmoid(jnp.dot(a.astype(bf), ea_w2[...],
                               preferred_element_type=f32) + ea_b2[...])
    we = w * e
    zr = (hr + we).astype(bf)
    zc = (hc + we).astype(bf)
    z = (jnp.dot(zr, cl_w1a[...], preferred_element_type=f32)
         + jnp.dot(zc, cl_w1b[...], preferred_element_type=f32) + cl_b1[...])
    z = jax.nn.relu(z)
    z = jax.nn.relu(jnp.dot(z.astype(bf), cl_w2[...],
                            preferred_element_type=f32) + cl_b2[...])
    out_ref[...] = (jnp.dot(z.astype(bf), cl_w3[...],
                            preferred_element_type=f32) + cl_b3[...])


def _edge_head(hr, hc, ea, wts):
    row_spec = pl.BlockSpec((_EB, D), lambda i: (i, 0))
    full = lambda a: pl.BlockSpec(a.shape, lambda i: (0,) * a.ndim)
    return pl.pallas_call(
        _edge_body,
        grid=(E // _EB,),
        in_specs=[row_spec, row_spec, row_spec] + [full(w) for w in wts],
        out_specs=pl.BlockSpec((_EB, 2), lambda i: (i, 0)),
        out_shape=jax.ShapeDtypeStruct((E, 2), jnp.float32),
    )(hr, hc, ea, *wts)


def kernel(x, edge_attr, params, edge_index):
    p = params
    src0 = edge_index[0]
    dst0 = edge_index[1]
    dst3 = dst0.reshape(_NW, _NCH, _CH)

    cnt = _sc_degree(dst0)

    h, y, dinv = _encoder(
        cnt, x,
        p['ne_W1'].T, p['ne_b1'][None], p['ne_W2'].T, p['ne_b2'][None],
        p['conv_W'][0].T)

    zeros_nd = jnp.zeros((_NP, D), jnp.float32)
    for i in range(NUM_LAYERS):
        parts = _sc_scatter(y, src0, dst3, zeros_nd)
        res = None
        if i > 0:
            res = (h, p['res_W'][i - 1].T, p['res_b'][i - 1][None])
        w_next = p['conv_W'][i + 1].T if i + 1 < NUM_LAYERS else None
        outs = _layer(parts, y, dinv,
                      p['conv_b'][i][None], p['bn_g'][i][None],
                      p['bn_b'][i][None], res=res, w_next=w_next)
        if w_next is not None:
            h, y = outs
        else:
            (h,) = outs

    hr, hc = _sc_edge_gather(h, src0, dst0)

    bf = jnp.bfloat16
    ea_w1t = p['ea_W1'].T
    cl_w1t = p['cl_W1'].T
    wts = [
        p['ee_W1'].T.astype(bf), p['ee_b1'][None],
        p['ee_W2'].T.astype(bf), p['ee_b2'][None],
        ea_w1t[:D].astype(bf), ea_w1t[D:2 * D].astype(bf),
        ea_w1t[2 * D:].astype(bf), p['ea_b1'][None],
        p['ea_W2'].T.astype(bf), p['ea_b2'][None],
        cl_w1t[:D].astype(bf), cl_w1t[D:].astype(bf), p['cl_b1'][None],
        p['cl_W2'].T.astype(bf), p['cl_b2'][None],
        p['cl_W3'].T.astype(bf), p['cl_b3'][None],
    ]
    return _edge_head(hr, hc, edge_attr, wts)


# speedup vs baseline: 1.1633x; 1.1633x over previous
import functools

import jax
import jax.numpy as jnp
from jax import lax
from jax.experimental import pallas as pl
from jax.experimental.pallas import tpu as pltpu
from jax.experimental.pallas import tpu_sc as plsc

N = 10000
E = 320000
D = 128
NUM_LAYERS = 4

_NC = 2
_NS = 16
_NW = _NC * _NS
_EPW = E // _NW
_CH = 80
_NCH = _EPW // _CH
_NP = 10240
_NPS = _NP // _NS

_sc_mesh = plsc.VectorSubcoreMesh(core_axis_name="c", subcore_axis_name="s")


def _deg_body(dst3_hbm, ones_hbm, zeros_hbm, out_hbm, didx_v, ones_v,
              s0, s1, s2, s3, s4, acc_sh):
    c = lax.axis_index("c")
    s = lax.axis_index("s")
    wid = c * _NS + s
    rows = pl.ds(s * _NPS, _NPS)
    pltpu.sync_copy(zeros_hbm.at[rows], acc_sh.at[rows])
    pltpu.sync_copy(ones_hbm, ones_v)
    pltpu.sync_copy(dst3_hbm.at[wid], didx_v)
    plsc.subcore_barrier()
    sems = [s0, s1, s2, s3, s4]

    def sstart(j, k):
        pltpu.async_copy(ones_v, acc_sh.at[didx_v.at[j]], sems[k], add=True)

    def swait(j, k):
        pltpu.make_async_copy(ones_v, acc_sh.at[didx_v.at[j]],
                              sems[k]).wait()

    for k in range(5):
        sstart(k, k)

    def step(i, carry):
        j0 = 5 * i
        for k in range(5):
            swait(j0 - 5 + k, k)
            sstart(j0 + k, k)
        return carry

    lax.fori_loop(1, _NCH // 5, step, 0)
    for k in range(5):
        swait(_NCH - 5 + k, k)
    plsc.subcore_barrier()
    pltpu.sync_copy(acc_sh.at[rows], out_hbm.at[c, rows])


def _sc_degree(dst3):
    ones = jnp.ones((_CH, D), jnp.float32)
    zeros = jnp.zeros((_NP, D), jnp.float32)
    return pl.kernel(
        _deg_body,
        out_type=jax.ShapeDtypeStruct((_NC, _NP, D), jnp.float32),
        mesh=_sc_mesh,
        scratch_types=[
            pltpu.VMEM((_NCH, _CH), jnp.int32),
            pltpu.VMEM((_CH, D), jnp.float32),
            pltpu.SemaphoreType.DMA,
            pltpu.SemaphoreType.DMA,
            pltpu.SemaphoreType.DMA,
            pltpu.SemaphoreType.DMA,
            pltpu.SemaphoreType.DMA,
            pltpu.VMEM_SHARED((_NP, D), jnp.float32),
        ],
    )(dst3, ones, zeros)


def _scat_body(y_hbm, src_hbm, dst3_hbm, zeros_hbm, out_hbm,
               sidx_v, didx_v, rows0, rows1, sem0, sem1, acc_sh):
    c = lax.axis_index("c")
    s = lax.axis_index("s")
    wid = c * _NS + s
    rows = pl.ds(s * _NPS, _NPS)
    pltpu.sync_copy(zeros_hbm.at[rows], acc_sh.at[rows])
    pltpu.sync_copy(src_hbm.at[pl.ds(wid * _EPW, _EPW)], sidx_v)
    pltpu.sync_copy(dst3_hbm.at[wid], didx_v)
    plsc.subcore_barrier()

    def gstart(j, buf, sem):
        pltpu.async_copy(y_hbm.at[sidx_v.at[pl.ds(j * _CH, _CH)]], buf, sem)

    def gwait(j, buf, sem):
        pltpu.make_async_copy(
            y_hbm.at[sidx_v.at[pl.ds(j * _CH, _CH)]], buf, sem).wait()

    def scat(j, buf):
        pltpu.sync_copy(buf, acc_sh.at[didx_v.at[j]], add=True)

    gstart(0, rows0, sem0)

    def step(i, carry):
        j0 = 2 * i
        gstart(j0 + 1, rows1, sem1)
        gwait(j0, rows0, sem0)
        scat(j0, rows0)
        gstart(j0 + 2, rows0, sem0)
        gwait(j0 + 1, rows1, sem1)
        scat(j0 + 1, rows1)
        return carry

    lax.fori_loop(0, (_NCH - 1) // 2, step, 0)
    gwait(_NCH - 1, rows0, sem0)
    scat(_NCH - 1, rows0)
    plsc.subcore_barrier()
    pltpu.sync_copy(acc_sh.at[rows], out_hbm.at[c, rows])


def _sc_scatter(y, src0, dst3, zeros_nd):
    return pl.kernel(
        _scat_body,
        out_type=jax.ShapeDtypeStruct((_NC, _NP, D), jnp.float32),
        mesh=_sc_mesh,
        scratch_types=[
            pltpu.VMEM((_EPW,), jnp.int32),
            pltpu.VMEM((_NCH, _CH), jnp.int32),
            pltpu.VMEM((_CH, D), jnp.float32),
            pltpu.VMEM((_CH, D), jnp.float32),
            pltpu.SemaphoreType.DMA,
            pltpu.SemaphoreType.DMA,
            pltpu.VMEM_SHARED((_NP, D), jnp.float32),
        ],
    )(y, src0, dst3, zeros_nd)


def _gath_body(epw, ch, nch,
               h_hbm, src_hbm, dst_hbm, hr_hbm, hc_hbm,
               sidx_v, didx_v,
               a0, a1, a2, a3, b0, b1, b2, b3,
               sa0, sa1, sa2, sa3, sb0, sb1, sb2, sb3):
    c = lax.axis_index("c")
    s = lax.axis_index("s")
    wid = c * _NS + s
    base = wid * epw
    pltpu.sync_copy(src_hbm.at[pl.ds(base, epw)], sidx_v)
    pltpu.sync_copy(dst_hbm.at[pl.ds(base, epw)], didx_v)
    abufs = [a0, a1, a2, a3]
    bbufs = [b0, b1, b2, b3]
    asems = [sa0, sa1, sa2, sa3]
    bsems = [sb0, sb1, sb2, sb3]

    def ga(j, k, idx_v, bufs, sems):
        pltpu.async_copy(h_hbm.at[idx_v.at[pl.ds(j * ch, ch)]],
                         bufs[k], sems[k])

    def gw(j, k, idx_v, bufs, sems):
        pltpu.make_async_copy(h_hbm.at[idx_v.at[pl.ds(j * ch, ch)]],
                              bufs[k], sems[k]).wait()

    def wr(j, buf, out):
        pltpu.sync_copy(buf, out.at[pl.ds(base + j * ch, ch)])

    for j in range(3):
        ga(j, j, sidx_v, abufs, asems)
        ga(j, j, didx_v, bbufs, bsems)

    def handle(j, k, do_issue):
        if do_issue:
            kn = (k + 3) % 4
            ga(j + 3, kn, sidx_v, abufs, asems)
            ga(j + 3, kn, didx_v, bbufs, bsems)
        gw(j, k, sidx_v, abufs, asems)
        wr(j, abufs[k], hr_hbm)
        gw(j, k, didx_v, bbufs, bsems)
        wr(j, bbufs[k], hc_hbm)

    def step(i, carry):
        j0 = 4 * i
        for k in range(4):
            handle(j0 + k, k, True)
        return carry

    nq = (nch - 5) // 4
    lax.fori_loop(0, nq, step, 0)
    for j in range(4 * nq, nch):
        handle(j, j % 4, j + 3 < nch)


def _sc_edge_gather(h, src_part, dst_part, e_part, ch):
    epw = e_part // _NW
    nch = epw // ch
    return pl.kernel(
        functools.partial(_gath_body, epw, ch, nch),
        out_type=(
            jax.ShapeDtypeStruct((e_part, D), jnp.float32),
            jax.ShapeDtypeStruct((e_part, D), jnp.float32),
        ),
        mesh=_sc_mesh,
        scratch_types=(
            [pltpu.VMEM((epw,), jnp.int32)] * 2
            + [pltpu.VMEM((ch, D), jnp.float32)] * 8
            + [pltpu.SemaphoreType.DMA] * 8
        ),
    )(h, src_part, dst_part)


def _enc_h_body(x_ref, w1_ref, b1_ref, w2_ref, b2_ref, h_ref):
    h = jnp.dot(x_ref[...], w1_ref[...], preferred_element_type=jnp.float32)
    h = jax.nn.relu(h + b1_ref[...])
    h_ref[...] = (jnp.dot(h, w2_ref[...], preferred_element_type=jnp.float32)
                  + b2_ref[...])


def _enc_h(x, w1t, b1, w2t, b2):
    return pl.pallas_call(
        _enc_h_body,
        out_shape=jax.ShapeDtypeStruct((N, D), jnp.float32),
    )(x, w1t, b1, w2t, b2)


def _enc_y_body(cnt_ref, h_ref, w0_ref, y_ref, dinv_ref):
    cnt = cnt_ref[...]
    deg = cnt[0, :N, 0:1] + cnt[1, :N, 0:1] + 1.0
    dinv = jax.lax.rsqrt(deg)
    dinv_ref[...] = dinv
    xw = jnp.dot(h_ref[...], w0_ref[...], preferred_element_type=jnp.float32)
    y_ref[...] = xw * dinv


def _enc_y(cnt, h, w0t):
    return pl.pallas_call(
        _enc_y_body,
        out_shape=(
            jax.ShapeDtypeStruct((N, D), jnp.float32),
            jax.ShapeDtypeStruct((N, 1), jnp.float32),
        ),
    )(cnt, h, w0t)


def _layer_body(has_res, has_next, *refs):
    it = iter(refs)
    parts = next(it)
    y = next(it); dinv = next(it)
    conv_b = next(it); bn_g = next(it); bn_b = next(it)
    if has_res:
        h_res = next(it); res_wt = next(it); res_b = next(it)
    if has_next:
        w_next = next(it)
    h_out = next(it)
    if has_next:
        y_next = next(it)

    p = parts[...]
    agg = (p[0, :N] + p[1, :N] + y[...]) * dinv[...] + conv_b[...]
    mu = jnp.mean(agg, axis=0, keepdims=True)
    var = jnp.mean((agg - mu) ** 2, axis=0, keepdims=True)
    hbn = (agg - mu) * jax.lax.rsqrt(var + 1e-5) * bn_g[...] + bn_b[...]
    h = jax.nn.relu(hbn)
    if has_res:
        h = h + jnp.dot(h_res[...], res_wt[...],
                        preferred_element_type=jnp.float32) + res_b[...]
    h_out[...] = h
    if has_next:
        y_next[...] = jnp.dot(h, w_next[...],
                              preferred_element_type=jnp.float32) * dinv[...]


def _layer(parts, y, dinv, conv_b, bn_g, bn_b, res=None, w_next=None):
    has_res = res is not None
    has_next = w_next is not None
    outs = [jax.ShapeDtypeStruct((N, D), jnp.float32)]
    if has_next:
        outs.append(jax.ShapeDtypeStruct((N, D), jnp.float32))
    args = [parts, y, dinv, conv_b, bn_g, bn_b]
    if has_res:
        args += list(res)
    if has_next:
        args.append(w_next)
    return pl.pallas_call(
        functools.partial(_layer_body, has_res, has_next),
        out_shape=tuple(outs),
    )(*args)


_EB = 2560


def _edge_body(hr_ref, hc_ref, ea_ref,
               ee_w1, ee_b1, ee_w2, ee_b2,
               ea_w1a, ea_w1b, ea_w1c, ea_b1, ea_w2, ea_b2,
               cl_w1a, cl_w1b, cl_b1, cl_w2, cl_b2, cl_w3, cl_b3,
               out_ref):
    hr = hr_ref[...]
    hc = hc_ref[...]
    ea = ea_ref[...]
    f32 = jnp.float32
    e = jax.nn.relu(jnp.dot(ea, ee_w1[...], preferred_element_type=f32)
                    + ee_b1[...])
    e = jnp.dot(e, ee_w2[...], preferred_element_type=f32) + ee_b2[...]
    a = (jnp.dot(hr, ea_w1a[...], preferred_element_type=f32)
         + jnp.dot(hc, ea_w1b[...], preferred_element_type=f32)
         + jnp.dot(ea, ea_w1c[...], preferred_element_type=f32) + ea_b1[...])
    a = jax.nn.relu(a)
    w = jax.nn.sigmoid(jnp.dot(a, ea_w2[...], preferred_element_type=f32)
                       + ea_b2[...])
    we = w * e
    zr = hr + we
    zc = hc + we
    z = (jnp.dot(zr, cl_w1a[...], preferred_element_type=f32)
         + jnp.dot(zc, cl_w1b[...], preferred_element_type=f32) + cl_b1[...])
    z = jax.nn.relu(z)
    z = jax.nn.relu(jnp.dot(z, cl_w2[...], preferred_element_type=f32)
                    + cl_b2[...])
    out_ref[...] = (jnp.dot(z, cl_w3[...], preferred_element_type=f32)
                    + cl_b3[...])


def _edge_head(hr, hc, ea, wts, e_part, off_blocks):
    row_spec = pl.BlockSpec((_EB, D), lambda i: (i, 0))
    ea_spec = pl.BlockSpec((_EB, D), lambda i: (i + off_blocks, 0))
    full = lambda a: pl.BlockSpec(a.shape, lambda i: (0,) * a.ndim)
    return pl.pallas_call(
        _edge_body,
        grid=(e_part // _EB,),
        in_specs=[row_spec, row_spec, ea_spec] + [full(w) for w in wts],
        out_specs=pl.BlockSpec((_EB, 2), lambda i: (i, 0)),
        out_shape=jax.ShapeDtypeStruct((e_part, 2), jnp.float32),
    )(hr, hc, ea, *wts)


def kernel(x, edge_attr, params, edge_index):
    p = params
    src0 = edge_index[0]
    dst0 = edge_index[1]
    dst3 = dst0.reshape(_NW, _NCH, _CH)

    cnt = _sc_degree(dst3)

    h = _enc_h(x, p['ne_W1'].T, p['ne_b1'][None], p['ne_W2'].T,
               p['ne_b2'][None])
    y, dinv = _enc_y(cnt, h, p['conv_W'][0].T)

    zeros_nd = jnp.zeros((_NP, D), jnp.float32)
    for i in range(NUM_LAYERS):
        parts = _sc_scatter(y, src0, dst3, zeros_nd)
        res = None
        if i > 0:
            res = (h, p['res_W'][i - 1].T, p['res_b'][i - 1][None])
        w_next = p['conv_W'][i + 1].T if i + 1 < NUM_LAYERS else None
        outs = _layer(parts, y, dinv,
                      p['conv_b'][i][None], p['bn_g'][i][None],
                      p['bn_b'][i][None], res=res, w_next=w_next)
        if w_next is not None:
            h, y = outs
        else:
            (h,) = outs

    ea_w1t = p['ea_W1'].T
    cl_w1t = p['cl_W1'].T
    wts = [
        p['ee_W1'].T, p['ee_b1'][None], p['ee_W2'].T, p['ee_b2'][None],
        ea_w1t[:D], ea_w1t[D:2 * D], ea_w1t[2 * D:], p['ea_b1'][None],
        p['ea_W2'].T, p['ea_b2'][None],
        cl_w1t[:D], cl_w1t[D:], p['cl_b1'][None],
        p['cl_W2'].T, p['cl_b2'][None], p['cl_W3'].T, p['cl_b3'][None],
    ]
    e0 = 62 * _NW * _CH
    e1 = E - e0
    hr0, hc0 = _sc_edge_gather(h, src0[:e0], dst0[:e0], e0, _CH)
    hr1, hc1 = _sc_edge_gather(h, src0[e0:], dst0[e0:], e1, _CH)
    out0 = _edge_head(hr0, hc0, edge_attr, wts, e0, 0)
    out1 = _edge_head(hr1, hc1, edge_attr, wts, e1, e0 // _EB)
    return jnp.concatenate([out0, out1], axis=0)


# speedup vs baseline: 1.1834x; 1.0173x over previous
import functools

import jax
import jax.numpy as jnp
from jax import lax
from jax.experimental import pallas as pl
from jax.experimental.pallas import tpu as pltpu
from jax.experimental.pallas import tpu_sc as plsc

N = 10000
E = 320000
D = 128
NUM_LAYERS = 4

_NC = 2
_NS = 16
_NW = _NC * _NS
_EPW = E // _NW
_CH = 80
_NCH = _EPW // _CH
_NP = 10240
_NPS = _NP // _NS

_sc_mesh = plsc.VectorSubcoreMesh(core_axis_name="c", subcore_axis_name="s")


def _deg_body(dst3_hbm, ones_hbm, zeros_hbm, out_hbm, didx_v, ones_v,
              s0, s1, s2, s3, s4, acc_sh):
    c = lax.axis_index("c")
    s = lax.axis_index("s")
    wid = c * _NS + s
    rows = pl.ds(s * _NPS, _NPS)
    pltpu.sync_copy(zeros_hbm.at[rows], acc_sh.at[rows])
    pltpu.sync_copy(ones_hbm, ones_v)
    pltpu.sync_copy(dst3_hbm.at[wid], didx_v)
    plsc.subcore_barrier()
    sems = [s0, s1, s2, s3, s4]

    def sstart(j, k):
        pltpu.async_copy(ones_v, acc_sh.at[didx_v.at[j]], sems[k], add=True)

    def swait(j, k):
        pltpu.make_async_copy(ones_v, acc_sh.at[didx_v.at[j]],
                              sems[k]).wait()

    for k in range(5):
        sstart(k, k)

    def step(i, carry):
        j0 = 5 * i
        for k in range(5):
            swait(j0 - 5 + k, k)
            sstart(j0 + k, k)
        return carry

    lax.fori_loop(1, _NCH // 5, step, 0)
    for k in range(5):
        swait(_NCH - 5 + k, k)
    plsc.subcore_barrier()
    pltpu.sync_copy(acc_sh.at[rows], out_hbm.at[c, rows])


def _sc_degree(dst3):
    ones = jnp.ones((_CH, D), jnp.float32)
    zeros = jnp.zeros((_NP, D), jnp.float32)
    return pl.kernel(
        _deg_body,
        out_type=jax.ShapeDtypeStruct((_NC, _NP, D), jnp.float32),
        mesh=_sc_mesh,
        scratch_types=[
            pltpu.VMEM((_NCH, _CH), jnp.int32),
            pltpu.VMEM((_CH, D), jnp.float32),
            pltpu.SemaphoreType.DMA,
            pltpu.SemaphoreType.DMA,
            pltpu.SemaphoreType.DMA,
            pltpu.SemaphoreType.DMA,
            pltpu.SemaphoreType.DMA,
            pltpu.VMEM_SHARED((_NP, D), jnp.float32),
        ],
    )(dst3, ones, zeros)


def _scat_body(y_hbm, src_hbm, dst3_hbm, zeros_hbm, out_hbm,
               sidx_v, didx_v, rows0, rows1, sem0, sem1, acc_sh):
    c = lax.axis_index("c")
    s = lax.axis_index("s")
    wid = c * _NS + s
    rows = pl.ds(s * _NPS, _NPS)
    pltpu.sync_copy(zeros_hbm.at[rows], acc_sh.at[rows])
    pltpu.sync_copy(src_hbm.at[pl.ds(wid * _EPW, _EPW)], sidx_v)
    pltpu.sync_copy(dst3_hbm.at[wid], didx_v)
    plsc.subcore_barrier()

    def gstart(j, buf, sem):
        pltpu.async_copy(y_hbm.at[sidx_v.at[pl.ds(j * _CH, _CH)]], buf, sem)

    def gwait(j, buf, sem):
        pltpu.make_async_copy(
            y_hbm.at[sidx_v.at[pl.ds(j * _CH, _CH)]], buf, sem).wait()

    def scat(j, buf):
        pltpu.sync_copy(buf, acc_sh.at[didx_v.at[j]], add=True)

    gstart(0, rows0, sem0)

    def step(i, carry):
        j0 = 2 * i
        gstart(j0 + 1, rows1, sem1)
        gwait(j0, rows0, sem0)
        scat(j0, rows0)
        gstart(j0 + 2, rows0, sem0)
        gwait(j0 + 1, rows1, sem1)
        scat(j0 + 1, rows1)
        return carry

    lax.fori_loop(0, (_NCH - 1) // 2, step, 0)
    gwait(_NCH - 1, rows0, sem0)
    scat(_NCH - 1, rows0)
    plsc.subcore_barrier()
    pltpu.sync_copy(acc_sh.at[rows], out_hbm.at[c, rows])


def _sc_scatter(y, src0, dst3, zeros_nd):
    return pl.kernel(
        _scat_body,
        out_type=jax.ShapeDtypeStruct((_NC, _NP, D), jnp.float32),
        mesh=_sc_mesh,
        scratch_types=[
            pltpu.VMEM((_EPW,), jnp.int32),
            pltpu.VMEM((_NCH, _CH), jnp.int32),
            pltpu.VMEM((_CH, D), jnp.float32),
            pltpu.VMEM((_CH, D), jnp.float32),
            pltpu.SemaphoreType.DMA,
            pltpu.SemaphoreType.DMA,
            pltpu.VMEM_SHARED((_NP, D), jnp.float32),
        ],
    )(y, src0, dst3, zeros_nd)


def _gath_body(epw, ch, nch,
               h_hbm, src_hbm, dst_hbm, hr_hbm, hc_hbm,
               sidx_v, didx_v,
               a0, a1, a2, a3, b0, b1, b2, b3,
               sa0, sa1, sa2, sa3, sb0, sb1, sb2, sb3):
    c = lax.axis_index("c")
    s = lax.axis_index("s")
    wid = c * _NS + s
    base = wid * epw
    pltpu.sync_copy(src_hbm.at[pl.ds(base, epw)], sidx_v)
    pltpu.sync_copy(dst_hbm.at[pl.ds(base, epw)], didx_v)
    abufs = [a0, a1, a2, a3]
    bbufs = [b0, b1, b2, b3]
    asems = [sa0, sa1, sa2, sa3]
    bsems = [sb0, sb1, sb2, sb3]

    def ga(j, k, idx_v, bufs, sems):
        pltpu.async_copy(h_hbm.at[idx_v.at[pl.ds(j * ch, ch)]],
                         bufs[k], sems[k])

    def gw(j, k, idx_v, bufs, sems):
        pltpu.make_async_copy(h_hbm.at[idx_v.at[pl.ds(j * ch, ch)]],
                              bufs[k], sems[k]).wait()

    def wr(j, buf, out):
        pltpu.sync_copy(buf, out.at[pl.ds(base + j * ch, ch)])

    for j in range(3):
        ga(j, j, sidx_v, abufs, asems)
        ga(j, j, didx_v, bbufs, bsems)

    def handle(j, k, do_issue):
        if do_issue:
            kn = (k + 3) % 4
            ga(j + 3, kn, sidx_v, abufs, asems)
            ga(j + 3, kn, didx_v, bbufs, bsems)
        gw(j, k, sidx_v, abufs, asems)
        wr(j, abufs[k], hr_hbm)
        gw(j, k, didx_v, bbufs, bsems)
        wr(j, bbufs[k], hc_hbm)

    def step(i, carry):
        j0 = 4 * i
        for k in range(4):
            handle(j0 + k, k, True)
        return carry

    nq = (nch - 5) // 4
    lax.fori_loop(0, nq, step, 0)
    for j in range(4 * nq, nch):
        handle(j, j % 4, j + 3 < nch)


def _sc_edge_gather(h, src_part, dst_part, e_part, ch):
    epw = e_part // _NW
    nch = epw // ch
    return pl.kernel(
        functools.partial(_gath_body, epw, ch, nch),
        out_type=(
            jax.ShapeDtypeStruct((e_part, D), jnp.float32),
            jax.ShapeDtypeStruct((e_part, D), jnp.float32),
        ),
        mesh=_sc_mesh,
        scratch_types=(
            [pltpu.VMEM((epw,), jnp.int32)] * 2
            + [pltpu.VMEM((ch, D), jnp.float32)] * 8
            + [pltpu.SemaphoreType.DMA] * 8
        ),
    )(h, src_part, dst_part)


def _enc_h_body(x_ref, w1_ref, b1_ref, w2_ref, b2_ref, h_ref):
    h = jnp.dot(x_ref[...], w1_ref[...], preferred_element_type=jnp.float32)
    h = jax.nn.relu(h + b1_ref[...])
    h_ref[...] = (jnp.dot(h, w2_ref[...], preferred_element_type=jnp.float32)
                  + b2_ref[...])


def _enc_h(x, w1t, b1, w2t, b2):
    return pl.pallas_call(
        _enc_h_body,
        out_shape=jax.ShapeDtypeStruct((N, D), jnp.float32),
    )(x, w1t, b1, w2t, b2)


def _enc_y_body(cnt_ref, h_ref, w0_ref, y_ref, dinv_ref):
    cnt = cnt_ref[...]
    deg = cnt[0, :N, 0:1] + cnt[1, :N, 0:1] + 1.0
    dinv = jax.lax.rsqrt(deg)
    dinv_ref[...] = dinv
    xw = jnp.dot(h_ref[...], w0_ref[...], preferred_element_type=jnp.float32)
    y_ref[...] = xw * dinv


def _enc_y(cnt, h, w0t):
    return pl.pallas_call(
        _enc_y_body,
        out_shape=(
            jax.ShapeDtypeStruct((N, D), jnp.float32),
            jax.ShapeDtypeStruct((N, 1), jnp.float32),
        ),
    )(cnt, h, w0t)


def _layer_body(has_res, has_next, *refs):
    it = iter(refs)
    parts = next(it)
    y = next(it); dinv = next(it)
    conv_b = next(it); bn_g = next(it); bn_b = next(it)
    if has_res:
        h_res = next(it); res_wt = next(it); res_b = next(it)
    if has_next:
        w_next = next(it)
    h_out = next(it)
    if has_next:
        y_next = next(it)

    p = parts[...]
    agg = (p[0, :N] + p[1, :N] + y[...]) * dinv[...] + conv_b[...]
    mu = jnp.mean(agg, axis=0, keepdims=True)
    var = jnp.mean((agg - mu) ** 2, axis=0, keepdims=True)
    hbn = (agg - mu) * jax.lax.rsqrt(var + 1e-5) * bn_g[...] + bn_b[...]
    h = jax.nn.relu(hbn)
    if has_res:
        h = h + jnp.dot(h_res[...], res_wt[...],
                        preferred_element_type=jnp.float32) + res_b[...]
    h_out[...] = h
    if has_next:
        y_next[...] = jnp.dot(h, w_next[...],
                              preferred_element_type=jnp.float32) * dinv[...]


def _layer(parts, y, dinv, conv_b, bn_g, bn_b, res=None, w_next=None):
    has_res = res is not None
    has_next = w_next is not None
    outs = [jax.ShapeDtypeStruct((N, D), jnp.float32)]
    if has_next:
        outs.append(jax.ShapeDtypeStruct((N, D), jnp.float32))
    args = [parts, y, dinv, conv_b, bn_g, bn_b]
    if has_res:
        args += list(res)
    if has_next:
        args.append(w_next)
    return pl.pallas_call(
        functools.partial(_layer_body, has_res, has_next),
        out_shape=tuple(outs),
    )(*args)


_EB = 8000


def _edge_body(hr_ref, hc_ref, ea_ref,
               ee_w1, ee_b1, ee_w2, ee_b2,
               ea_w1a, ea_w1b, ea_w1c, ea_b1, ea_w2, ea_b2,
               cl_w1a, cl_w1b, cl_b1, cl_w2, cl_b2, cl_w3, cl_b3,
               out_ref):
    hr = hr_ref[...]
    hc = hc_ref[...]
    ea = ea_ref[...]
    f32 = jnp.float32
    e = jax.nn.relu(jnp.dot(ea, ee_w1[...], preferred_element_type=f32)
                    + ee_b1[...])
    e = jnp.dot(e, ee_w2[...], preferred_element_type=f32) + ee_b2[...]
    a = (jnp.dot(hr, ea_w1a[...], preferred_element_type=f32)
         + jnp.dot(hc, ea_w1b[...], preferred_element_type=f32)
         + jnp.dot(ea, ea_w1c[...], preferred_element_type=f32) + ea_b1[...])
    a = jax.nn.relu(a)
    w = jax.nn.sigmoid(jnp.dot(a, ea_w2[...], preferred_element_type=f32)
                       + ea_b2[...])
    we = w * e
    zr = hr + we
    zc = hc + we
    z = (jnp.dot(zr, cl_w1a[...], preferred_element_type=f32)
         + jnp.dot(zc, cl_w1b[...], preferred_element_type=f32) + cl_b1[...])
    z = jax.nn.relu(z)
    z = jax.nn.relu(jnp.dot(z, cl_w2[...], preferred_element_type=f32)
                    + cl_b2[...])
    out_ref[...] = (jnp.dot(z, cl_w3[...], preferred_element_type=f32)
                    + cl_b3[...])


def _edge_head(hr, hc, ea, wts, e_part, off_blocks):
    row_spec = pl.BlockSpec((_EB, D), lambda i: (i, 0))
    ea_spec = pl.BlockSpec((_EB, D), lambda i: (i + off_blocks, 0))
    full = lambda a: pl.BlockSpec(a.shape, lambda i: (0,) * a.ndim)
    return pl.pallas_call(
        _edge_body,
        grid=(e_part // _EB,),
        in_specs=[row_spec, row_spec, ea_spec] + [full(w) for w in wts],
        out_specs=pl.BlockSpec((_EB, 2), lambda i: (i, 0)),
        out_shape=jax.ShapeDtypeStruct((e_part, 2), jnp.float32),
    )(hr, hc, ea, *wts)


def kernel(x, edge_attr, params, edge_index):
    p = params
    src0 = edge_index[0]
    dst0 = edge_index[1]
    dst3 = dst0.reshape(_NW, _NCH, _CH)

    cnt = _sc_degree(dst3)

    h = _enc_h(x, p['ne_W1'].T, p['ne_b1'][None], p['ne_W2'].T,
               p['ne_b2'][None])
    y, dinv = _enc_y(cnt, h, p['conv_W'][0].T)

    zeros_nd = jnp.zeros((_NP, D), jnp.float32)
    for i in range(NUM_LAYERS):
        parts = _sc_scatter(y, src0, dst3, zeros_nd)
        res = None
        if i > 0:
            res = (h, p['res_W'][i - 1].T, p['res_b'][i - 1][None])
        w_next = p['conv_W'][i + 1].T if i + 1 < NUM_LAYERS else None
        outs = _layer(parts, y, dinv,
                      p['conv_b'][i][None], p['bn_g'][i][None],
                      p['bn_b'][i][None], res=res, w_next=w_next)
        if w_next is not None:
            h, y = outs
        else:
            (h,) = outs

    ea_w1t = p['ea_W1'].T
    cl_w1t = p['cl_W1'].T
    wts = [
        p['ee_W1'].T, p['ee_b1'][None], p['ee_W2'].T, p['ee_b2'][None],
        ea_w1t[:D], ea_w1t[D:2 * D], ea_w1t[2 * D:], p['ea_b1'][None],
        p['ea_W2'].T, p['ea_b2'][None],
        cl_w1t[:D], cl_w1t[D:], p['cl_b1'][None],
        p['cl_W2'].T, p['cl_b2'][None], p['cl_W3'].T, p['cl_b3'][None],
    ]
    eh = E // 2
    hr0, hc0 = _sc_edge_gather(h, src0[:eh], dst0[:eh], eh, 40)
    hr1, hc1 = _sc_edge_gather(h, src0[eh:], dst0[eh:], eh, 40)
    out0 = _edge_head(hr0, hc0, edge_attr, wts, eh, 0)
    out1 = _edge_head(hr1, hc1, edge_attr, wts, eh, eh // _EB)
    return jnp.concatenate([out0, out1], axis=0)
